# baseline (speedup 1.0000x reference)
import functools

import jax
import jax.numpy as jnp
from jax import lax
from jax.experimental import pallas as pl
from jax.experimental.pallas import tpu as pltpu
from jax.experimental.pallas import tpu_sc as plsc

KK = 8
RROWS = 8192
CCOLS = 8192
NC, NS, L = 2, 16, 16
NW = NC * NS
RPW = RROWS // NW
NCHUNK = CCOLS // L
CAP = NCHUNK
BROWS = 2
NBLK = RPW // BROWS
NBUF = 4
PADC = 1 << 14
PADB = 1 << 15
NEG = float("-inf")

_mesh = plsc.VectorSubcoreMesh(
    core_axis_name="c", subcore_axis_name="s", num_cores=NC, num_subcores=NS
)


@functools.partial(
    pl.kernel,
    out_type=jax.ShapeDtypeStruct((RROWS * KK,), jnp.float32),
    mesh=_mesh,
    compiler_params=pltpu.CompilerParams(needs_layout_passes=False),
    scratch_types=[
        *[pltpu.VMEM((BROWS * CCOLS,), jnp.float32) for _ in range(NBUF)],
        pltpu.VMEM((L * CAP,), jnp.int32),
        pltpu.VMEM((L,), jnp.int32),
        pltpu.VMEM((2 * L,), jnp.float32),
        pltpu.VMEM((2 * L,), jnp.int32),
        pltpu.VMEM((RPW * KK + L,), jnp.float32),
        *[pltpu.SemaphoreType.DMA for _ in range(NBUF)],
    ],
)
def _kmax_sc(x_hbm, out_hbm, buf0, buf1, buf2, buf3, colbuf, plbuf, mbv, mbi,
             outbuf, sem0, sem1, sem2, sem3):
    bufs = (buf0, buf1, buf2, buf3)
    sems = (sem0, sem1, sem2, sem3)
    wid = lax.axis_index("s") * NC + lax.axis_index("c")
    row0 = wid * RPW
    lane = lax.iota(jnp.int32, L)

    def process(rowbuf, rr):
        @plsc.parallel_loop(0, NCHUNK, unroll=16,
                            carry=jnp.full((L,), NEG, jnp.float32))
        def acc(i, a):
            return jnp.maximum(a, rowbuf[pl.ds(i * L, L)])
        sk, _ = plsc.sort_key_val(acc, acc)
        t = jnp.max(jnp.where(lane == (L - 1 - KK), sk, NEG))

        @plsc.parallel_loop(0, NCHUNK, unroll=8,
                            carry=(jnp.zeros((L,), jnp.int32), lane))
        def p2res(i, carry):
            plcnt, col = carry
            v = rowbuf[pl.ds(i * L, L)]
            m = v >= t
            plsc.store_scatter(colbuf, [lane + (plcnt << 4)], col, mask=m)
            return plcnt + jnp.where(m, 1, 0).astype(jnp.int32), col + L

        (plcnt, _) = p2res
        plbuf[pl.ds(0, L)] = plcnt
        maxc = jnp.max(plcnt)

        mbv[pl.ds(0, L)] = jnp.full((L,), NEG, jnp.float32)
        mbi[pl.ds(0, L)] = PADB + lane

        def pc(u, carry2):
            j = lax.shift_right_logical(u, 1)
            half = jnp.bitwise_and(u, 1)
            src_lane = jnp.bitwise_and(lane, KK - 1) + half * KK
            plc_g = plsc.load_gather(plbuf, [src_lane])
            valid = (lane >= KK) & (j < plc_g)
            bidx = src_lane + lax.shift_left(j, 4)
            cols_raw = plsc.load_gather(colbuf, [bidx])
            gidx = jnp.where(valid, cols_raw, 0)
            gv = plsc.load_gather(rowbuf, [gidx])
            mbv[pl.ds(KK, L)] = jnp.full((L,), NEG, jnp.float32)
            mbi[pl.ds(KK, L)] = PADC + lane
            plsc.store_compressed(mbv.at[pl.ds(KK, L)], gv, mask=valid)
            plsc.store_compressed(mbi.at[pl.ds(KK, L)], gidx, mask=valid)
            comb_v = mbv[pl.ds(0, L)]
            comb_i = mbi[pl.ds(0, L)]
            rank = jnp.zeros((L,), jnp.int32)
            for r in range(1, L):
                perm = (lane + r) & (L - 1)
                rv = plsc.load_gather(mbv, [perm])
                ri = plsc.load_gather(mbi, [perm])
                gt = (rv > comb_v) | ((rv == comb_v) & (ri < comb_i))
                rank = rank + jnp.where(gt, 1, 0).astype(jnp.int32)
            keep = rank < KK
            plsc.store_compressed(mbv.at[pl.ds(0, L)], comb_v, mask=keep)
            plsc.store_compressed(mbi.at[pl.ds(0, L)], comb_i, mask=keep)
            return carry2

        lax.fori_loop(0, 2 * maxc, pc, 0)

        mbv[pl.ds(KK, L)] = jnp.full((L,), NEG, jnp.float32)
        mbi[pl.ds(KK, L)] = PADB + lane
        bi = mbi[pl.ds(0, L)]
        bv = mbv[pl.ds(0, L)]
        posn = jnp.zeros((L,), jnp.int32)
        for r in range(1, L):
            perm = (lane + r) & (L - 1)
            ri = plsc.load_gather(mbi, [perm])
            posn = posn + jnp.where(ri < bi, 1, 0).astype(jnp.int32)
        plsc.store_scatter(outbuf, [rr * KK + posn], bv, mask=lane < KK)

    BC = BROWS * CCOLS

    def blk_base(g):
        return (row0 + g * BROWS) * CCOLS

    for u in range(NBUF):
        pltpu.async_copy(x_hbm.at[pl.ds(blk_base(u), BC)], bufs[u], sems[u])

    def blk(j, carry):
        for u in range(NBUF):
            g = j * NBUF + u
            pltpu.make_async_copy(x_hbm.at[pl.ds(blk_base(g), BC)],
                                  bufs[u], sems[u]).wait()
            for tr in range(BROWS):
                process(bufs[u].at[pl.ds(tr * CCOLS, CCOLS)],
                        g * BROWS + tr)
            nxt = jnp.where(g + NBUF < NBLK, g + NBUF, 0)
            pltpu.async_copy(x_hbm.at[pl.ds(blk_base(nxt), BC)],
                             bufs[u], sems[u])
        return carry

    lax.fori_loop(0, NBLK // NBUF, blk, 0)
    for u in range(NBUF):
        pltpu.make_async_copy(x_hbm.at[pl.ds(blk_base(0), BC)],
                              bufs[u], sems[u]).wait()

    pltpu.sync_copy(outbuf.at[pl.ds(0, RPW * KK)],
                    out_hbm.at[pl.ds(row0 * KK, RPW * KK)])


def kernel(x):
    out = _kmax_sc(x.reshape(RROWS * CCOLS))
    return out.reshape(8, 1024, KK)


# speedup vs baseline: 1.5797x; 1.5797x over previous
import functools

import jax
import jax.numpy as jnp
from jax import lax
from jax.experimental import pallas as pl
from jax.experimental.pallas import tpu as pltpu
from jax.experimental.pallas import tpu_sc as plsc

KK = 8
RROWS = 8192
CCOLS = 8192
NC, NS, L = 2, 16, 16
NW = NC * NS
RPW = RROWS // NW
NCHUNK = CCOLS // L
CAP = NCHUNK
BROWS = 4
NBLK = RPW // BROWS
PADC = 1 << 14
PADB = 1 << 15
NEG = float("-inf")

_mesh = plsc.VectorSubcoreMesh(
    core_axis_name="c", subcore_axis_name="s", num_cores=NC, num_subcores=NS
)


@functools.partial(
    pl.kernel,
    out_type=jax.ShapeDtypeStruct((RROWS * KK,), jnp.float32),
    mesh=_mesh,
    compiler_params=pltpu.CompilerParams(needs_layout_passes=False,
                                         use_tc_tiling_on_sc=True),
    scratch_types=[
        pltpu.VMEM((CCOLS,), jnp.float32),
        pltpu.VMEM((CCOLS,), jnp.float32),
        pltpu.VMEM((CCOLS,), jnp.float32),
        pltpu.VMEM((CCOLS,), jnp.float32),
        pltpu.VMEM((L * CAP,), jnp.int32),
        pltpu.VMEM((L,), jnp.int32),
        pltpu.VMEM((2 * L,), jnp.float32),
        pltpu.VMEM((2 * L,), jnp.int32),
        pltpu.VMEM((RPW * KK + L,), jnp.float32),
        pltpu.SemaphoreType.DMA,
        pltpu.SemaphoreType.DMA,
        pltpu.SemaphoreType.DMA,
        pltpu.SemaphoreType.DMA,
    ],
)
def _kmax_sc(x_hbm, out_hbm, buf0, buf1, buf2, buf3, colbuf, plbuf, mbv, mbi,
             outbuf, sem0, sem1, sem2, sem3):
    bufs = (buf0, buf1, buf2, buf3)
    sems = (sem0, sem1, sem2, sem3)
    wid = lax.axis_index("s") * NC + lax.axis_index("c")
    row0 = wid * RPW
    lane = lax.iota(jnp.int32, L)

    def process(rowbuf, rr):
        @plsc.parallel_loop(0, NCHUNK, unroll=16,
                            carry=jnp.full((L,), NEG, jnp.float32))
        def acc(i, a):
            return jnp.maximum(a, rowbuf[pl.ds(i * L, L)])
        sk, _ = plsc.sort_key_val(acc, acc)
        t = jnp.max(jnp.where(lane == (L - 1 - KK), sk, NEG))

        @plsc.parallel_loop(0, NCHUNK, unroll=8,
                            carry=(jnp.zeros((L,), jnp.int32), lane))
        def p2res(i, carry):
            plcnt, col = carry
            v = rowbuf[pl.ds(i * L, L)]
            m = v >= t
            plsc.store_scatter(colbuf, [lane + (plcnt << 4)], col, mask=m)
            return plcnt + jnp.where(m, 1, 0).astype(jnp.int32), col + L

        (plcnt, _) = p2res
        plbuf[pl.ds(0, L)] = plcnt
        maxc = jnp.max(plcnt)

        mbv[pl.ds(0, L)] = jnp.full((L,), NEG, jnp.float32)
        mbi[pl.ds(0, L)] = PADB + lane

        def pc(u, carry2):
            j = lax.shift_right_logical(u, 1)
            half = jnp.bitwise_and(u, 1)
            src_lane = jnp.bitwise_and(lane, KK - 1) + half * KK
            plc_g = plsc.load_gather(plbuf, [src_lane])
            valid = (lane >= KK) & (j < plc_g)
            bidx = src_lane + lax.shift_left(j, 4)
            cols_raw = plsc.load_gather(colbuf, [bidx])
            gidx = jnp.where(valid, cols_raw, 0)
            gv = plsc.load_gather(rowbuf, [gidx])
            mbv[pl.ds(KK, L)] = jnp.full((L,), NEG, jnp.float32)
            mbi[pl.ds(KK, L)] = PADC + lane
            plsc.store_compressed(mbv.at[pl.ds(KK, L)], gv, mask=valid)
            plsc.store_compressed(mbi.at[pl.ds(KK, L)], gidx, mask=valid)
            comb_v = mbv[pl.ds(0, L)]
            comb_i = mbi[pl.ds(0, L)]
            rank = jnp.zeros((L,), jnp.int32)
            for r in range(1, L):
                perm = (lane + r) & (L - 1)
                rv = plsc.load_gather(mbv, [perm])
                ri = plsc.load_gather(mbi, [perm])
                gt = (rv > comb_v) | ((rv == comb_v) & (ri < comb_i))
                rank = rank + jnp.where(gt, 1, 0).astype(jnp.int32)
            keep = rank < KK
            plsc.store_compressed(mbv.at[pl.ds(0, L)], comb_v, mask=keep)
            plsc.store_compressed(mbi.at[pl.ds(0, L)], comb_i, mask=keep)
            return carry2

        lax.fori_loop(0, 2 * maxc, pc, 0)

        mbv[pl.ds(KK, L)] = jnp.full((L,), NEG, jnp.float32)
        mbi[pl.ds(KK, L)] = PADB + lane
        bi = mbi[pl.ds(0, L)]
        bv = mbv[pl.ds(0, L)]
        posn = jnp.zeros((L,), jnp.int32)
        for r in range(1, L):
            perm = (lane + r) & (L - 1)
            ri = plsc.load_gather(mbi, [perm])
            posn = posn + jnp.where(ri < bi, 1, 0).astype(jnp.int32)
        plsc.store_scatter(outbuf, [rr * KK + posn], bv, mask=lane < KK)

    bpage = wid // 4
    d0 = (wid % 4) * RPW

    for u in range(4):
        pltpu.async_copy(x_hbm.at[bpage, d0 + u], bufs[u], sems[u])

    def blk(j, carry):
        for u in range(4):
            rr = j * 4 + u
            pltpu.make_async_copy(x_hbm.at[bpage, d0 + rr],
                                  bufs[u], sems[u]).wait()
            process(bufs[u], rr)
            nxt = jnp.where(rr + 4 < RPW, d0 + rr + 4, d0)
            pltpu.async_copy(x_hbm.at[bpage, nxt], bufs[u], sems[u])
        return carry

    lax.fori_loop(0, RPW // 4, blk, 0)
    for u in range(4):
        pltpu.make_async_copy(x_hbm.at[bpage, d0], bufs[u], sems[u]).wait()

    pltpu.sync_copy(outbuf.at[pl.ds(0, RPW * KK)],
                    out_hbm.at[pl.ds(row0 * KK, RPW * KK)])


def kernel(x):
    out = _kmax_sc(x)
    return out.reshape(8, 1024, KK)


# speedup vs baseline: 1.5937x; 1.0089x over previous
import functools

import jax
import jax.numpy as jnp
from jax import lax
from jax.experimental import pallas as pl
from jax.experimental.pallas import tpu as pltpu
from jax.experimental.pallas import tpu_sc as plsc

KK = 8
RROWS = 8192
CCOLS = 8192
NC, NS, L = 2, 16, 16
NW = NC * NS
RPW = RROWS // NW
NCHUNK = CCOLS // L
CAP = NCHUNK
BROWS = 4
NBLK = RPW // BROWS
PADC = 1 << 14
PADB = 1 << 15
NEG = float("-inf")

_mesh = plsc.VectorSubcoreMesh(
    core_axis_name="c", subcore_axis_name="s", num_cores=NC, num_subcores=NS
)


@functools.partial(
    pl.kernel,
    out_type=jax.ShapeDtypeStruct((RROWS * KK,), jnp.float32),
    mesh=_mesh,
    compiler_params=pltpu.CompilerParams(needs_layout_passes=False,
                                         use_tc_tiling_on_sc=True),
    scratch_types=[
        pltpu.VMEM((CCOLS,), jnp.float32),
        pltpu.VMEM((CCOLS,), jnp.float32),
        pltpu.VMEM((CCOLS,), jnp.float32),
        pltpu.VMEM((CCOLS,), jnp.float32),
        pltpu.VMEM((L * CAP,), jnp.int32),
        pltpu.VMEM((L,), jnp.int32),
        pltpu.VMEM((2 * L,), jnp.float32),
        pltpu.VMEM((2 * L,), jnp.int32),
        pltpu.VMEM((RPW * KK + L,), jnp.float32),
        pltpu.SemaphoreType.DMA,
        pltpu.SemaphoreType.DMA,
        pltpu.SemaphoreType.DMA,
        pltpu.SemaphoreType.DMA,
    ],
)
def _kmax_sc(x_hbm, out_hbm, buf0, buf1, buf2, buf3, colbuf, plbuf, mbv, mbi,
             outbuf, sem0, sem1, sem2, sem3):
    bufs = (buf0, buf1, buf2, buf3)
    sems = (sem0, sem1, sem2, sem3)
    wid = lax.axis_index("s") * NC + lax.axis_index("c")
    row0 = wid * RPW
    lane = lax.iota(jnp.int32, L)

    def process(rowbuf, rr):
        @plsc.parallel_loop(0, NCHUNK, unroll=16,
                            carry=jnp.full((L,), NEG, jnp.float32))
        def acc(i, a):
            return jnp.maximum(a, rowbuf[pl.ds(i * L, L)])
        sk, _ = plsc.sort_key_val(acc, acc)
        t = jnp.max(jnp.where(lane == (L - 1 - KK), sk, NEG))

        @plsc.parallel_loop(0, NCHUNK, unroll=16,
                            carry=(jnp.zeros((L,), jnp.int32), lane))
        def p2res(i, carry):
            plcnt, col = carry
            v = rowbuf[pl.ds(i * L, L)]
            m = v >= t
            plsc.store_scatter(colbuf, [lane + (plcnt << 4)], col, mask=m)
            return plcnt + jnp.where(m, 1, 0).astype(jnp.int32), col + L

        (plcnt, _) = p2res
        plbuf[pl.ds(0, L)] = plcnt
        maxc = jnp.max(plcnt)

        mbv[pl.ds(0, L)] = jnp.full((L,), NEG, jnp.float32)
        mbi[pl.ds(0, L)] = PADB + lane

        def pc(u, carry2):
            j = lax.shift_right_logical(u, 1)
            half = jnp.bitwise_and(u, 1)
            src_lane = jnp.bitwise_and(lane, KK - 1) + half * KK
            plc_g = plsc.load_gather(plbuf, [src_lane])
            valid = (lane >= KK) & (j < plc_g)
            bidx = src_lane + lax.shift_left(j, 4)
            cols_raw = plsc.load_gather(colbuf, [bidx])
            gidx = jnp.where(valid, cols_raw, 0)
            gv = plsc.load_gather(rowbuf, [gidx])
            mbv[pl.ds(KK, L)] = jnp.full((L,), NEG, jnp.float32)
            mbi[pl.ds(KK, L)] = PADC + lane
            plsc.store_compressed(mbv.at[pl.ds(KK, L)], gv, mask=valid)
            plsc.store_compressed(mbi.at[pl.ds(KK, L)], gidx, mask=valid)
            comb_v = mbv[pl.ds(0, L)]
            comb_i = mbi[pl.ds(0, L)]
            rank = jnp.zeros((L,), jnp.int32)
            for r in range(1, L):
                perm = (lane + r) & (L - 1)
                rv = plsc.load_gather(mbv, [perm])
                ri = plsc.load_gather(mbi, [perm])
                gt = (rv > comb_v) | ((rv == comb_v) & (ri < comb_i))
                rank = rank + jnp.where(gt, 1, 0).astype(jnp.int32)
            keep = rank < KK
            plsc.store_compressed(mbv.at[pl.ds(0, L)], comb_v, mask=keep)
            plsc.store_compressed(mbi.at[pl.ds(0, L)], comb_i, mask=keep)
            return carry2

        lax.fori_loop(0, 2 * maxc, pc, 0)

        mbv[pl.ds(KK, L)] = jnp.full((L,), NEG, jnp.float32)
        mbi[pl.ds(KK, L)] = PADB + lane
        bi = mbi[pl.ds(0, L)]
        bv = mbv[pl.ds(0, L)]
        posn = jnp.zeros((L,), jnp.int32)
        for r in range(1, L):
            perm = (lane + r) & (L - 1)
            ri = plsc.load_gather(mbi, [perm])
            posn = posn + jnp.where(ri < bi, 1, 0).astype(jnp.int32)
        plsc.store_scatter(outbuf, [rr * KK + posn], bv, mask=lane < KK)

    bpage = wid // 4
    d0 = (wid % 4) * RPW

    for u in range(4):
        pltpu.async_copy(x_hbm.at[bpage, d0 + u], bufs[u], sems[u])

    def blk(j, carry):
        for u in range(4):
            rr = j * 4 + u
            pltpu.make_async_copy(x_hbm.at[bpage, d0 + rr],
                                  bufs[u], sems[u]).wait()
            process(bufs[u], rr)
            nxt = jnp.where(rr + 4 < RPW, d0 + rr + 4, d0)
            pltpu.async_copy(x_hbm.at[bpage, nxt], bufs[u], sems[u])
        return carry

    lax.fori_loop(0, RPW // 4, blk, 0)
    for u in range(4):
        pltpu.make_async_copy(x_hbm.at[bpage, d0], bufs[u], sems[u]).wait()

    pltpu.sync_copy(outbuf.at[pl.ds(0, RPW * KK)],
                    out_hbm.at[pl.ds(row0 * KK, RPW * KK)])


def kernel(x):
    out = _kmax_sc(x)
    return out.reshape(8, 1024, KK)


# speedup vs baseline: 1.5942x; 1.0003x over previous
import functools

import jax
import jax.numpy as jnp
from jax import lax
from jax.experimental import pallas as pl
from jax.experimental.pallas import tpu as pltpu
from jax.experimental.pallas import tpu_sc as plsc

KK = 8
RROWS = 8192
CCOLS = 8192
NC, NS, L = 2, 16, 16
NW = NC * NS
RPW = RROWS // NW
NCHUNK = CCOLS // L
CAP = NCHUNK
PADC = 1 << 14
PADB = 1 << 15
NEG = float("-inf")

_mesh = plsc.VectorSubcoreMesh(
    core_axis_name="c", subcore_axis_name="s", num_cores=NC, num_subcores=NS
)


@functools.partial(
    pl.kernel,
    out_type=jax.ShapeDtypeStruct((RROWS * KK,), jnp.float32),
    mesh=_mesh,
    compiler_params=pltpu.CompilerParams(needs_layout_passes=False,
                                         use_tc_tiling_on_sc=True),
    scratch_types=[
        pltpu.VMEM((CCOLS,), jnp.float32),
        pltpu.VMEM((CCOLS,), jnp.float32),
        pltpu.VMEM((CCOLS,), jnp.float32),
        pltpu.VMEM((CCOLS,), jnp.float32),
        pltpu.VMEM((L * CAP,), jnp.int32),
        pltpu.VMEM((L,), jnp.int32),
        pltpu.VMEM((2 * L,), jnp.float32),
        pltpu.VMEM((2 * L,), jnp.int32),
        pltpu.VMEM((RPW * KK + L,), jnp.float32),
        pltpu.SemaphoreType.DMA,
        pltpu.SemaphoreType.DMA,
        pltpu.SemaphoreType.DMA,
        pltpu.SemaphoreType.DMA,
    ],
)
def _kmax_sc(x_hbm, out_hbm, buf0, buf1, buf2, buf3, colbuf, plbuf, mbv, mbi,
             outbuf, sem0, sem1, sem2, sem3):
    bufs = (buf0, buf1, buf2, buf3)
    sems = (sem0, sem1, sem2, sem3)
    wid = lax.axis_index("s") * NC + lax.axis_index("c")
    row0 = wid * RPW
    lane = lax.iota(jnp.int32, L)

    def process(rowbuf, rr):
        @plsc.parallel_loop(0, NCHUNK, unroll=16,
                            carry=jnp.full((L,), NEG, jnp.float32))
        def acc(i, a):
            return jnp.maximum(a, rowbuf[pl.ds(i * L, L)])
        sk, _ = plsc.sort_key_val(acc, acc)
        t = jnp.max(jnp.where(lane == (L - 1 - KK), sk, NEG))

        @plsc.parallel_loop(0, NCHUNK, unroll=16,
                            carry=(jnp.zeros((L,), jnp.int32), lane))
        def p2res(i, carry):
            plcnt, col = carry
            v = rowbuf[pl.ds(i * L, L)]
            m = v >= t
            plsc.store_scatter(colbuf, [lane + (plcnt << 4)], col, mask=m)
            return plcnt + jnp.where(m, 1, 0).astype(jnp.int32), col + L

        (plcnt, _) = p2res
        plbuf[pl.ds(0, L)] = plcnt
        maxc = jnp.max(plcnt)

        mbv[pl.ds(0, L)] = jnp.full((L,), NEG, jnp.float32)
        mbi[pl.ds(0, L)] = PADB + lane

        def pc(u, carry2):
            j = lax.shift_right_logical(u, 1)
            half = jnp.bitwise_and(u, 1)
            src_lane = jnp.bitwise_and(lane, KK - 1) + half * KK
            plc_g = plsc.load_gather(plbuf, [src_lane])
            valid = (lane >= KK) & (j < plc_g)
            bidx = src_lane + lax.shift_left(j, 4)
            cols_raw = plsc.load_gather(colbuf, [bidx])
            gidx = jnp.where(valid, cols_raw, 0)
            gv = plsc.load_gather(rowbuf, [gidx])
            mbv[pl.ds(KK, L)] = jnp.full((L,), NEG, jnp.float32)
            mbi[pl.ds(KK, L)] = PADC + lane
            plsc.store_compressed(mbv.at[pl.ds(KK, L)], gv, mask=valid)
            plsc.store_compressed(mbi.at[pl.ds(KK, L)], gidx, mask=valid)
            comb_v = mbv[pl.ds(0, L)]
            comb_i = mbi[pl.ds(0, L)]
            rank = jnp.zeros((L,), jnp.int32)
            for r in range(1, L):
                perm = (lane + r) & (L - 1)
                rv = plsc.load_gather(mbv, [perm])
                ri = plsc.load_gather(mbi, [perm])
                gt = (rv > comb_v) | ((rv == comb_v) & (ri < comb_i))
                rank = rank + jnp.where(gt, 1, 0).astype(jnp.int32)
            keep = rank < KK
            plsc.store_compressed(mbv.at[pl.ds(0, L)], comb_v, mask=keep)
            plsc.store_compressed(mbi.at[pl.ds(0, L)], comb_i, mask=keep)
            return carry2

        lax.fori_loop(0, 2 * maxc, pc, 0)

        mbv[pl.ds(KK, L)] = jnp.full((L,), NEG, jnp.float32)
        mbi[pl.ds(KK, L)] = PADB + lane
        bi = mbi[pl.ds(0, L)]
        bv = mbv[pl.ds(0, L)]
        posn = jnp.zeros((L,), jnp.int32)
        for r in range(1, L):
            perm = (lane + r) & (L - 1)
            ri = plsc.load_gather(mbi, [perm])
            posn = posn + jnp.where(ri < bi, 1, 0).astype(jnp.int32)
        plsc.store_scatter(outbuf, [rr * KK + posn], bv, mask=lane < KK)

    bpage = wid // 4
    d0 = (wid % 4) * RPW

    for u in range(4):
        pltpu.async_copy(x_hbm.at[bpage, d0 + u], bufs[u], sems[u])

    def blk(j, carry):
        for u in range(4):
            rr = j * 4 + u
            pltpu.make_async_copy(x_hbm.at[bpage, d0 + rr],
                                  bufs[u], sems[u]).wait()
            process(bufs[u], rr)
            nxt = jnp.where(rr + 4 < RPW, d0 + rr + 4, d0)
            pltpu.async_copy(x_hbm.at[bpage, nxt], bufs[u], sems[u])
        return carry

    lax.fori_loop(0, RPW // 4, blk, 0)
    for u in range(4):
        pltpu.make_async_copy(x_hbm.at[bpage, d0], bufs[u], sems[u]).wait()

    pltpu.sync_copy(outbuf.at[pl.ds(0, RPW * KK)],
                    out_hbm.at[pl.ds(row0 * KK, RPW * KK)])


def kernel(x):
    out = _kmax_sc(x)
    return out.reshape(8, 1024, KK)
